# Initial kernel scaffold; baseline (speedup 1.0000x reference)
import jax
import jax.numpy as jnp
from jax.experimental import pallas as pl


def kernel(x, edge_index, Wl0, bl0, Wr0, a0, Ws0, bs0, Wl1, bl1, Wr1, a1, Ws1, bs1):
    raise NotImplementedError("write your pallas kernel here")


# speedup vs baseline: 2.6549x; 2.6549x over previous
import jax
import jax.numpy as jnp
from jax import lax
from jax.experimental import pallas as pl
from jax.experimental.pallas import tpu as pltpu
from jax.experimental.pallas import tpu_sc as plsc

_N = 10000
_D = 128
_E = 320000

_NC = 2
_NS = 16
_NW = _NC * _NS
_C = 128
_NCH = 80
_EW = _C * _NCH
_EPAD = _NW * _EW
_NPAD = 10240
_RPT = _NPAD // _NS

_BR = 1024
_GRID = _NPAD // _BR


def _agg_body(h_hbm, src_hbm, dst_hbm, z_hbm, p_out, sidx, didx, rows, zbuf,
              acc, sem_s, sem_d, sem_g):
    c = lax.axis_index("c")
    s = lax.axis_index("s")
    base = (c * _NS + s) * _EW
    r0 = s * _RPT
    pltpu.sync_copy(z_hbm, zbuf)

    @pl.loop(0, _RPT // 16)
    def _z(k):
        rr = pl.multiple_of(r0 + k * 16, 16)
        pltpu.sync_copy(zbuf, acc.at[pl.ds(rr, 16)])

    plsc.subcore_barrier()

    @pl.loop(0, _NCH)
    def _chunk(j):
        off = pl.multiple_of(base + j * _C, _C)
        cs = pltpu.async_copy(src_hbm.at[pl.ds(off, _C)], sidx.at[0], sem_s)
        cd = pltpu.async_copy(dst_hbm.at[pl.ds(off, _C)], didx.at[0], sem_d)
        cs.wait()
        g = pltpu.async_copy(h_hbm.at[sidx.at[0]], rows.at[0], sem_g)
        cd.wait()
        g.wait()
        pltpu.sync_copy(rows.at[0], acc.at[didx.at[0]], add=True)

    plsc.subcore_barrier()

    @pl.loop(0, _RPT // _C)
    def _o(k):
        rr = pl.multiple_of(r0 + k * _C, _C)
        ro = pl.multiple_of(c * _NPAD + rr, _C)
        pltpu.sync_copy(acc.at[pl.ds(rr, _C)], rows.at[0])
        pltpu.sync_copy(rows.at[0], p_out.at[pl.ds(ro, _C)])


_agg = pl.kernel(
    _agg_body,
    out_type=jax.ShapeDtypeStruct((_NC * _NPAD, _D), jnp.float32),
    mesh=plsc.VectorSubcoreMesh(core_axis_name="c", subcore_axis_name="s"),
    scratch_types=[
        pltpu.VMEM((1, _C), jnp.int32),
        pltpu.VMEM((1, _C), jnp.int32),
        pltpu.VMEM((1, _C, _D), jnp.float32),
        pltpu.VMEM((16, _D), jnp.float32),
        pltpu.VMEM_SHARED((_NPAD, _D), jnp.float32),
        pltpu.SemaphoreType.DMA,
        pltpu.SemaphoreType.DMA,
        pltpu.SemaphoreType.DMA,
    ],
)


def _deg_body(dst_hbm, z_hbm, o_hbm, deg_out, didx, rows, zbuf, acc, sem_d):
    c = lax.axis_index("c")
    s = lax.axis_index("s")
    base = (c * _NS + s) * _EW
    r0 = s * _RPT
    pltpu.sync_copy(z_hbm, zbuf)
    pltpu.sync_copy(o_hbm, rows.at[0])

    @pl.loop(0, _RPT // 16)
    def _z(k):
        rr = pl.multiple_of(r0 + k * 16, 16)
        pltpu.sync_copy(zbuf, acc.at[pl.ds(rr, 16)])

    plsc.subcore_barrier()

    @pl.loop(0, _NCH)
    def _chunk(j):
        off = pl.multiple_of(base + j * _C, _C)
        pltpu.async_copy(dst_hbm.at[pl.ds(off, _C)], didx.at[0], sem_d).wait()
        pltpu.sync_copy(rows.at[0], acc.at[didx.at[0]], add=True)

    plsc.subcore_barrier()

    @pl.loop(0, _RPT // _C)
    def _o(k):
        rr = pl.multiple_of(r0 + k * _C, _C)
        ro = pl.multiple_of(c * _NPAD + rr, _C)
        pltpu.sync_copy(acc.at[pl.ds(rr, _C)], rows.at[0])
        pltpu.sync_copy(rows.at[0], deg_out.at[pl.ds(ro, _C)])


_deg = pl.kernel(
    _deg_body,
    out_type=jax.ShapeDtypeStruct((_NC * _NPAD, _D), jnp.float32),
    mesh=plsc.VectorSubcoreMesh(core_axis_name="c", subcore_axis_name="s"),
    scratch_types=[
        pltpu.VMEM((1, _C), jnp.int32),
        pltpu.VMEM((1, _C, _D), jnp.float32),
        pltpu.VMEM((16, _D), jnp.float32),
        pltpu.VMEM_SHARED((_NPAD, _D), jnp.float32),
        pltpu.SemaphoreType.DMA,
    ],
)


def _dense_body(p_ref, deg_ref, h_ref, xs_ref, wl_ref, bl_ref, wr_ref, a_ref,
                ws_ref, bs_ref, o_ref):
    agg = p_ref[0] + p_ref[1]
    deg = deg_ref[0] + deg_ref[1]
    inv = 1.0 / jnp.maximum(deg, 1.0)
    dims = (((1,), (1,)), ((), ()))
    z = lax.dot_general(agg * inv, wl_ref[...], dims,
                        preferred_element_type=jnp.float32)
    z = z + bl_ref[...][None, :] + lax.dot_general(
        h_ref[...], wr_ref[...], dims, preferred_element_type=jnp.float32)
    act = jnp.where(z >= 0, z, a_ref[...][None, :] * z)
    skip = lax.dot_general(xs_ref[...], ws_ref[...], dims,
                           preferred_element_type=jnp.float32)
    o_ref[...] = act + skip + bs_ref[...][None, :]


def _dense(p, degp, h, xs, wl, bl, wr, a, ws, bs):
    wspec = pl.BlockSpec((_D, _D), lambda i: (0, 0))
    bspec = pl.BlockSpec((_D,), lambda i: (0,))
    return pl.pallas_call(
        _dense_body,
        grid=(_GRID,),
        in_specs=[
            pl.BlockSpec((_NC, _BR, _D), lambda i: (0, i, 0)),
            pl.BlockSpec((_NC, _BR, _D), lambda i: (0, i, 0)),
            pl.BlockSpec((_BR, _D), lambda i: (i, 0)),
            pl.BlockSpec((_BR, _D), lambda i: (i, 0)),
            wspec, bspec, wspec, bspec, wspec, bspec,
        ],
        out_specs=pl.BlockSpec((_BR, _D), lambda i: (i, 0)),
        out_shape=jax.ShapeDtypeStruct((_NPAD, _D), jnp.float32),
    )(p, degp, h, xs, wl, bl, wr, a, ws, bs)


def kernel(x, edge_index, Wl0, bl0, Wr0, a0, Ws0, bs0, Wl1, bl1, Wr1, a1,
           Ws1, bs1):
    pad = _EPAD - _E
    src_p = jnp.concatenate([edge_index[0], jnp.zeros((pad,), jnp.int32)])
    dst_p = jnp.concatenate([edge_index[1], jnp.full((pad,), _N, jnp.int32)])
    z16 = jnp.zeros((16, _D), jnp.float32)
    ones = jnp.ones((_C, _D), jnp.float32)
    xp = jnp.concatenate([x, jnp.zeros((_NPAD - _N, _D), jnp.float32)])

    degp = _deg(dst_p, z16, ones).reshape(_NC, _NPAD, _D)
    p0 = _agg(xp, src_p, dst_p, z16).reshape(_NC, _NPAD, _D)
    h1 = _dense(p0, degp, xp, xp, Wl0, bl0, Wr0, a0, Ws0, bs0)
    p1 = _agg(h1, src_p, dst_p, z16).reshape(_NC, _NPAD, _D)
    h2 = _dense(p1, degp, h1, xp, Wl1, bl1, Wr1, a1, Ws1, bs1)
    return h2[:_N]


# speedup vs baseline: 3.4189x; 1.2878x over previous
import jax
import jax.numpy as jnp
from jax import lax
from jax.experimental import pallas as pl
from jax.experimental.pallas import tpu as pltpu
from jax.experimental.pallas import tpu_sc as plsc

_N = 10000
_D = 128
_E = 320000

_NC = 2
_NS = 16
_NW = _NC * _NS
_C = 128
_NCH = 80
_NP2 = _NCH // 2
_EW = _C * _NCH
_EPAD = _NW * _EW
_NPAD = 10240
_RPT = _NPAD // _NS

_BR = 1024
_GRID = _NPAD // _BR


def _agg_body(h_hbm, src_hbm, dst_hbm, z_hbm, p_out, sidx, didx, rows, zbuf,
              acc, ss0, ss1, sd0, sd1, sg0, sg1):
    c = lax.axis_index("c")
    s = lax.axis_index("s")
    base = (c * _NS + s) * _EW
    r0 = s * _RPT
    pltpu.sync_copy(z_hbm, zbuf)

    @pl.loop(0, _RPT // 64)
    def _z(k):
        rr = pl.multiple_of(r0 + k * 64, 64)
        pltpu.sync_copy(zbuf, acc.at[pl.ds(rr, 64)])

    plsc.subcore_barrier()

    c00 = pltpu.async_copy(src_hbm.at[pl.ds(base, _C)], sidx.at[0], ss0)
    c01 = pltpu.async_copy(dst_hbm.at[pl.ds(base, _C)], didx.at[0], sd0)
    c10 = pltpu.async_copy(src_hbm.at[pl.ds(base + _C, _C)], sidx.at[1], ss1)
    c11 = pltpu.async_copy(dst_hbm.at[pl.ds(base + _C, _C)], didx.at[1], sd1)

    @pl.loop(0, _NP2)
    def _pair(j):
        off = pl.multiple_of(base + j * (2 * _C), _C)
        pltpu.make_async_copy(src_hbm.at[pl.ds(off, _C)], sidx.at[0],
                              ss0).wait()
        g0 = pltpu.async_copy(h_hbm.at[sidx.at[0]], rows.at[0], sg0)
        pltpu.make_async_copy(src_hbm.at[pl.ds(off, _C)], sidx.at[1],
                              ss1).wait()
        g1 = pltpu.async_copy(h_hbm.at[sidx.at[1]], rows.at[1], sg1)
        g0.wait()
        pltpu.make_async_copy(dst_hbm.at[pl.ds(off, _C)], didx.at[0],
                              sd0).wait()
        pltpu.sync_copy(rows.at[0], acc.at[didx.at[0]], add=True)

        @pl.when(j + 1 < _NP2)
        def _pf0():
            nxt = pl.multiple_of(off + 2 * _C, _C)
            pltpu.async_copy(src_hbm.at[pl.ds(nxt, _C)], sidx.at[0], ss0)
            pltpu.async_copy(dst_hbm.at[pl.ds(nxt, _C)], didx.at[0], sd0)

        g1.wait()
        pltpu.make_async_copy(dst_hbm.at[pl.ds(off, _C)], didx.at[1],
                              sd1).wait()
        pltpu.sync_copy(rows.at[1], acc.at[didx.at[1]], add=True)

        @pl.when(j + 1 < _NP2)
        def _pf1():
            nxt = pl.multiple_of(off + 3 * _C, _C)
            pltpu.async_copy(src_hbm.at[pl.ds(nxt, _C)], sidx.at[1], ss1)
            pltpu.async_copy(dst_hbm.at[pl.ds(nxt, _C)], didx.at[1], sd1)

    plsc.subcore_barrier()

    @pl.loop(0, _RPT // _C)
    def _o(k):
        rr = pl.multiple_of(r0 + k * _C, _C)
        ro = pl.multiple_of(c * _NPAD + rr, _C)
        pltpu.sync_copy(acc.at[pl.ds(rr, _C)], rows.at[0])
        pltpu.sync_copy(rows.at[0], p_out.at[pl.ds(ro, _C)])


_agg = pl.kernel(
    _agg_body,
    out_type=jax.ShapeDtypeStruct((_NC * _NPAD, _D), jnp.float32),
    mesh=plsc.VectorSubcoreMesh(core_axis_name="c", subcore_axis_name="s"),
    scratch_types=[
        pltpu.VMEM((2, _C), jnp.int32),
        pltpu.VMEM((2, _C), jnp.int32),
        pltpu.VMEM((2, _C, _D), jnp.float32),
        pltpu.VMEM((64, _D), jnp.float32),
        pltpu.VMEM_SHARED((_NPAD, _D), jnp.float32),
        pltpu.SemaphoreType.DMA,
        pltpu.SemaphoreType.DMA,
        pltpu.SemaphoreType.DMA,
        pltpu.SemaphoreType.DMA,
        pltpu.SemaphoreType.DMA,
        pltpu.SemaphoreType.DMA,
    ],
)


def _deg_body(dst_hbm, z_hbm, o_hbm, deg_out, didx, rows, zbuf, acc,
              sd0, sd1):
    c = lax.axis_index("c")
    s = lax.axis_index("s")
    base = (c * _NS + s) * _EW
    r0 = s * _RPT
    pltpu.sync_copy(z_hbm, zbuf)
    pltpu.sync_copy(o_hbm, rows.at[0])

    @pl.loop(0, _RPT // 64)
    def _z(k):
        rr = pl.multiple_of(r0 + k * 64, 64)
        pltpu.sync_copy(zbuf, acc.at[pl.ds(rr, 64)])

    plsc.subcore_barrier()

    c0 = pltpu.async_copy(dst_hbm.at[pl.ds(base, _C)], didx.at[0], sd0)
    c1 = pltpu.async_copy(dst_hbm.at[pl.ds(base + _C, _C)], didx.at[1], sd1)

    @pl.loop(0, _NP2)
    def _pair(j):
        off = pl.multiple_of(base + j * (2 * _C), _C)
        pltpu.make_async_copy(dst_hbm.at[pl.ds(off, _C)], didx.at[0],
                              sd0).wait()
        pltpu.sync_copy(rows.at[0], acc.at[didx.at[0]], add=True)

        @pl.when(j + 1 < _NP2)
        def _pf0():
            nxt = pl.multiple_of(off + 2 * _C, _C)
            pltpu.async_copy(dst_hbm.at[pl.ds(nxt, _C)], didx.at[0], sd0)

        pltpu.make_async_copy(dst_hbm.at[pl.ds(off, _C)], didx.at[1],
                              sd1).wait()
        pltpu.sync_copy(rows.at[0], acc.at[didx.at[1]], add=True)

        @pl.when(j + 1 < _NP2)
        def _pf1():
            nxt = pl.multiple_of(off + 3 * _C, _C)
            pltpu.async_copy(dst_hbm.at[pl.ds(nxt, _C)], didx.at[1], sd1)

    plsc.subcore_barrier()

    @pl.loop(0, _RPT // _C)
    def _o(k):
        rr = pl.multiple_of(r0 + k * _C, _C)
        ro = pl.multiple_of(c * _NPAD + rr, _C)
        pltpu.sync_copy(acc.at[pl.ds(rr, _C)], rows.at[0])
        pltpu.sync_copy(rows.at[0], deg_out.at[pl.ds(ro, _C)])


_deg = pl.kernel(
    _deg_body,
    out_type=jax.ShapeDtypeStruct((_NC * _NPAD, _D), jnp.float32),
    mesh=plsc.VectorSubcoreMesh(core_axis_name="c", subcore_axis_name="s"),
    scratch_types=[
        pltpu.VMEM((2, _C), jnp.int32),
        pltpu.VMEM((1, _C, _D), jnp.float32),
        pltpu.VMEM((64, _D), jnp.float32),
        pltpu.VMEM_SHARED((_NPAD, _D), jnp.float32),
        pltpu.SemaphoreType.DMA,
        pltpu.SemaphoreType.DMA,
    ],
)


def _dense_body(p_ref, deg_ref, h_ref, xs_ref, wl_ref, bl_ref, wr_ref, a_ref,
                ws_ref, bs_ref, o_ref):
    agg = p_ref[0] + p_ref[1]
    deg = deg_ref[0] + deg_ref[1]
    inv = 1.0 / jnp.maximum(deg, 1.0)
    dims = (((1,), (1,)), ((), ()))
    z = lax.dot_general(agg * inv, wl_ref[...], dims,
                        preferred_element_type=jnp.float32)
    z = z + bl_ref[...][None, :] + lax.dot_general(
        h_ref[...], wr_ref[...], dims, preferred_element_type=jnp.float32)
    act = jnp.where(z >= 0, z, a_ref[...][None, :] * z)
    skip = lax.dot_general(xs_ref[...], ws_ref[...], dims,
                           preferred_element_type=jnp.float32)
    o_ref[...] = act + skip + bs_ref[...][None, :]


def _dense(p, degp, h, xs, wl, bl, wr, a, ws, bs):
    wspec = pl.BlockSpec((_D, _D), lambda i: (0, 0))
    bspec = pl.BlockSpec((_D,), lambda i: (0,))
    return pl.pallas_call(
        _dense_body,
        grid=(_GRID,),
        in_specs=[
            pl.BlockSpec((_NC, _BR, _D), lambda i: (0, i, 0)),
            pl.BlockSpec((_NC, _BR, _D), lambda i: (0, i, 0)),
            pl.BlockSpec((_BR, _D), lambda i: (i, 0)),
            pl.BlockSpec((_BR, _D), lambda i: (i, 0)),
            wspec, bspec, wspec, bspec, wspec, bspec,
        ],
        out_specs=pl.BlockSpec((_BR, _D), lambda i: (i, 0)),
        out_shape=jax.ShapeDtypeStruct((_NPAD, _D), jnp.float32),
    )(p, degp, h, xs, wl, bl, wr, a, ws, bs)


def kernel(x, edge_index, Wl0, bl0, Wr0, a0, Ws0, bs0, Wl1, bl1, Wr1, a1,
           Ws1, bs1):
    pad = _EPAD - _E
    src_p = jnp.concatenate([edge_index[0], jnp.zeros((pad,), jnp.int32)])
    dst_p = jnp.concatenate([edge_index[1], jnp.full((pad,), _N, jnp.int32)])
    zc = jnp.zeros((64, _D), jnp.float32)
    ones = jnp.ones((_C, _D), jnp.float32)
    xp = jnp.concatenate([x, jnp.zeros((_NPAD - _N, _D), jnp.float32)])

    degp = _deg(dst_p, zc, ones).reshape(_NC, _NPAD, _D)
    p0 = _agg(xp, src_p, dst_p, zc).reshape(_NC, _NPAD, _D)
    h1 = _dense(p0, degp, xp, xp, Wl0, bl0, Wr0, a0, Ws0, bs0)
    p1 = _agg(h1, src_p, dst_p, zc).reshape(_NC, _NPAD, _D)
    h2 = _dense(p1, degp, h1, xp, Wl1, bl1, Wr1, a1, Ws1, bs1)
    return h2[:_N]
